# Initial kernel scaffold; baseline (speedup 1.0000x reference)
import jax
import jax.numpy as jnp
from jax.experimental import pallas as pl


def kernel(x, edge_index, W, b, Wsrc, bsrc, Wtgt, btgt):
    raise NotImplementedError("write your pallas kernel here")


# speedup vs baseline: 3.5217x; 3.5217x over previous
import jax
import jax.numpy as jnp
from jax import lax
from jax.experimental import pallas as pl
from jax.experimental.pallas import tpu as pltpu
from jax.experimental.pallas import tpu_sc as plsc

N = 10000
E = 320000
D = 128
K = 4
KD = K * D

NC = 2
NS = 16
NW = NC * NS
NHALF = N // NC
NH = 5120
NHR = NH // 128
SL = E // NS
SCN = 2000
CAP = 12096
C = 64
RPT = NH // NS
CROWS = 48

_f32 = jnp.float32


def _prep_body(x_ref, w_ref, b_ref, wsrc_ref, bsrc_ref, wtgt_ref,
               btgt_ref, h0_ref, h1_ref, h2_ref, h3_ref, p_ref, self_ref):
    xb = x_ref[...]
    s = jnp.dot(xb, wsrc_ref[...].T,
                preferred_element_type=_f32) + bsrc_ref[...]
    t = jnp.dot(xb, wtgt_ref[...].T,
                preferred_element_type=_f32) + btgt_ref[...]
    es = jnp.exp(s)
    et = jnp.exp(t)
    bes = jax.lax.bitcast_convert_type(es, jnp.int32)
    bet = jax.lax.bitcast_convert_type(et, jnp.int32)
    hi = (bes + 0x8000) & jnp.int32(-65536)
    lo = jax.lax.shift_right_logical(bet + 0x8000, 16)
    p_ref[...] = hi | lo
    p = es * et
    psum = p[:, 0:1] + p[:, 1:2] + p[:, 2:3] + p[:, 3:4]
    q = p / psum
    acc = None
    for k, hk_ref in enumerate((h0_ref, h1_ref, h2_ref, h3_ref)):
        hk = jnp.dot(xb, w_ref[k * D:(k + 1) * D, :].T,
                     preferred_element_type=_f32) + b_ref[k:k + 1, :]
        hk_ref[...] = hk
        term = q[:, k:k + 1] * hk
        acc = term if acc is None else acc + term
    self_ref[...] = acc


def _prep(x, wcat, b, wsrc, bsrc, wtgt, btgt):
    BN = 400
    grid = (N // BN,)
    full = lambda shp: pl.BlockSpec(shp, lambda i: (0,) * len(shp))
    row_spec = pl.BlockSpec((BN, D), lambda i: (i, 0))
    return pl.pallas_call(
        _prep_body,
        grid=grid,
        in_specs=[
            row_spec,
            full((KD, D)), full((K, D)),
            full((K, D)), full((1, K)),
            full((K, D)), full((1, K)),
        ],
        out_specs=[
            row_spec, row_spec, row_spec, row_spec,
            pl.BlockSpec((BN, K), lambda i: (i, 0)),
            row_spec,
        ],
        out_shape=[
            jax.ShapeDtypeStruct((N, D), _f32),
            jax.ShapeDtypeStruct((N, D), _f32),
            jax.ShapeDtypeStruct((N, D), _f32),
            jax.ShapeDtypeStruct((N, D), _f32),
            jax.ShapeDtypeStruct((N, K), jnp.int32),
            jax.ShapeDtypeStruct((N, D), _f32),
        ],
    )(x, wcat, b, wsrc, bsrc, wtgt, btgt)


def _sc_body(dst_hbm, src_hbm, h0_hbm, h1_hbm, h2_hbm, h3_hbm, p_hbm,
             self_hbm, out_hbm,
             ebuf, sdst, dsti, srcc,
             rows, ptab, msg, qbuf, cntv, rowids, num_sp, cnt_sp, sem_h):
    cid = lax.axis_index("c")
    sid = lax.axis_index("s")
    lo = cid * NHALF
    hi_b = lo + NHALF
    eb = sid * SL
    h_hbms = (h0_hbm, h1_hbm, h2_hbm, h3_hbm)

    pltpu.sync_copy(p_hbm, ptab)

    z16f = jnp.zeros((16,), _f32)
    z16i = jnp.zeros((16,), jnp.int32)
    iota16 = lax.iota(jnp.int32, 16)

    def _zc(i, carry):
        cntv[i // 8, pl.ds((i % 8) * 16, 16)] = z16f
        return carry
    lax.fori_loop(0, CROWS * 8, _zc, 0)

    def _zbufs(i, carry):
        ebuf[pl.ds(i * 16, 16)] = z16i
        return carry
    lax.fori_loop(0, CAP // 16, _zbufs, 0)

    def _zr(i, carry):
        rows[i // 8, pl.ds((i % 8) * 16, 16)] = z16f
        return carry
    lax.fori_loop(0, 64, _zr, 0)

    def _ri(i, carry):
        rowids[pl.ds(i * 16, 16)] = iota16 + i * 16
        return carry
    lax.fori_loop(0, CROWS // 16, _ri, 0)

    def _zn(i, carry):
        pltpu.sync_copy(rows.at[pl.ds(0, 8)],
                        num_sp.at[pl.ds(sid * RPT + i * 8, 8)])
        return carry
    lax.fori_loop(0, RPT // 8, _zn, 0)

    @pl.when(sid < CROWS // 8)
    def _zcsp():
        pltpu.sync_copy(rows.at[pl.ds(0, 8)], cnt_sp.at[pl.ds(sid * 8, 8)])

    plsc.subcore_barrier()

    lov = jnp.full((16,), 0, jnp.int32) + lo
    hiv = jnp.full((16,), 0, jnp.int32) + hi_b

    def _scan(ci, nval):
        pltpu.sync_copy(dst_hbm.at[pl.ds(eb + ci * SCN, SCN)], sdst)

        def _inner(j, nv):
            dv = sdst[pl.ds(j * 16, 16)]
            m = (dv >= lov) & (dv < hiv)
            pos = (eb + ci * SCN + j * 16) + iota16
            plsc.store_compressed(ebuf.at[pl.ds(nv, 16)], pos, mask=m)
            c16 = plsc.all_reduce_population_count(m)
            return jnp.minimum(nv + c16[0], CAP - 32)
        return lax.fori_loop(0, SCN // 16, _inner, nval)

    nval = lax.fori_loop(0, SL // SCN, _scan, jnp.int32(0))

    himask = jnp.full((16,), -65536, jnp.int32)
    ones16 = jnp.ones((16,), _f32)
    nvalv = jnp.full((16,), 0, jnp.int32) + nval
    nh1 = jnp.full((16,), NHALF - 1, jnp.int32)

    def _chunk(i, carry):
        off = i * C
        idxsl = ebuf.at[pl.ds(off, C)]
        pltpu.async_copy(dst_hbm.at[idxsl], dsti, sem_h).wait()
        pltpu.async_copy(src_hbm.at[idxsl], srcc, sem_h).wait()

        def _loc(b, carry2):
            dv = dsti[pl.ds(b * 16, 16)] - lov
            dsti[pl.ds(b * 16, 16)] = jnp.minimum(jnp.maximum(dv, 0), nh1)
            return carry2
        lax.fori_loop(0, C // 16, _loc, 0)

        def _qblk(blk, carry2):
            dstv = dsti[pl.ds(blk * 16, 16)]
            srcv = srcc[pl.ds(blk * 16, 16)]
            valid = (off + blk * 16 + iota16) < nvalv
            plsc.addupdate_scatter(cntv, [dstv >> 7, dstv & 127], ones16,
                                   mask=valid)
            dstg4 = (dstv + lov) * 4
            srcg4 = srcv * 4
            p = []
            for k in range(K):
                gd = plsc.load_gather(ptab, [dstg4 + k])
                gs = plsc.load_gather(ptab, [srcg4 + k])
                esv = plsc.bitcast(gd & himask, _f32)
                etv = plsc.bitcast(lax.shift_left(gs, 16), _f32)
                p.append(esv * etv)
            rden = jnp.where(valid, 1.0 / (p[0] + p[1] + p[2] + p[3]), 0.0)
            for k in range(K):
                qbuf[k, pl.ds(blk * 16, 16)] = p[k] * rden
            return carry2
        lax.fori_loop(0, C // 16, _qblk, 0)

        for k in range(K):
            pltpu.async_copy(h_hbms[k].at[srcc], rows, sem_h).wait()

            def _mblk(blk, carry2, k=k):
                qv = qbuf[k, pl.ds(blk * 16, 16)]
                for l in range(16):
                    e = blk * 16 + l
                    qb = jnp.full((16,), qv[l], _f32)
                    for j in range(D // 16):
                        term = qb * rows[e, pl.ds(j * 16, 16)]
                        if k == 0:
                            msg[e, pl.ds(j * 16, 16)] = term
                        else:
                            msg[e, pl.ds(j * 16, 16)] = (
                                msg[e, pl.ds(j * 16, 16)] + term)
                return carry2
            lax.fori_loop(0, C // 16, _mblk, 0)

        pltpu.sync_copy(msg, num_sp.at[dsti], add=True)
        return carry

    nchunk = (nval + (C - 1)) // C
    lax.fori_loop(0, nchunk, _chunk, 0)

    pltpu.sync_copy(cntv, cnt_sp.at[rowids], add=True)
    plsc.subcore_barrier()

    pltpu.sync_copy(cnt_sp, cntv)

    def _fin(i, carry):
        base = sid * RPT + i * 16
        pltpu.sync_copy(num_sp.at[pl.ds(base, 16)], rows.at[pl.ds(0, 16)])
        pltpu.sync_copy(self_hbm.at[cid, pl.ds(base, 16)],
                        rows.at[pl.ds(16, 16)])
        cv = cntv[base >> 7, pl.ds(base & 127, 16)]
        rv = 1.0 / (cv + 1.0)
        for l in range(16):
            sc = jnp.full((16,), rv[l], _f32)
            for j in range(D // 16):
                msg[l, pl.ds(j * 16, 16)] = sc * (
                    rows[l, pl.ds(j * 16, 16)]
                    + rows[16 + l, pl.ds(j * 16, 16)])
        pltpu.sync_copy(msg.at[pl.ds(0, 16)],
                        out_hbm.at[cid, pl.ds(base, 16)])
        return carry
    lax.fori_loop(0, RPT // 16, _fin, 0)


def _sc_edges(dst, src, h0, h1, h2, h3, ptab, selfp):
    mesh = plsc.VectorSubcoreMesh(core_axis_name="c", subcore_axis_name="s",
                                  num_cores=NC, num_subcores=NS)
    f = pl.kernel(
        _sc_body,
        out_type=jax.ShapeDtypeStruct((NC, NH, D), _f32),
        mesh=mesh,
        compiler_params=pltpu.CompilerParams(needs_layout_passes=False),
        scratch_types=[
            pltpu.VMEM((CAP,), jnp.int32),
            pltpu.VMEM((SCN,), jnp.int32),
            pltpu.VMEM((C,), jnp.int32),
            pltpu.VMEM((C,), jnp.int32),
            pltpu.VMEM((C, D), _f32),
            pltpu.VMEM((N * K,), jnp.int32),
            pltpu.VMEM((C, D), _f32),
            pltpu.VMEM((K, C), _f32),
            pltpu.VMEM((CROWS, 128), _f32),
            pltpu.VMEM((CROWS,), jnp.int32),
            pltpu.VMEM_SHARED((NH, D), _f32),
            pltpu.VMEM_SHARED((CROWS, 128), _f32),
            pltpu.SemaphoreType.DMA,
        ],
    )
    return f(dst, src, h0, h1, h2, h3, ptab, selfp)


@jax.jit
def kernel(x, edge_index, W, b, Wsrc, bsrc, Wtgt, btgt):
    wcat = W.reshape(KD, D)
    h0, h1, h2, h3, ptab, selfmsg = _prep(x, wcat, b,
                                          Wsrc, bsrc.reshape(1, K),
                                          Wtgt, btgt.reshape(1, K))
    dst = edge_index[0]
    src = edge_index[1]
    selfp = jnp.zeros((NC, NH, D), _f32)
    selfp = selfp.at[0, :NHALF].set(selfmsg[:NHALF])
    selfp = selfp.at[1, :NHALF].set(selfmsg[NHALF:])
    o = _sc_edges(dst, src, h0, h1, h2, h3, ptab.reshape(N * K), selfp)
    return jnp.concatenate([o[0, :NHALF], o[1, :NHALF]], axis=0)


# speedup vs baseline: 4.2414x; 1.2043x over previous
import jax
import jax.numpy as jnp
from jax import lax
from jax.experimental import pallas as pl
from jax.experimental.pallas import tpu as pltpu
from jax.experimental.pallas import tpu_sc as plsc

N = 10000
E = 320000
D = 128
K = 4
KD = K * D

NC = 2
NS = 16
NW = NC * NS
NHALF = N // NC
NH = 5120
NHR = NH // 128
SL = E // NS
SCN = 2000
CAP = 11136
C = 48
RPT = NH // NS
CROWS = 48

_f32 = jnp.float32


def _prep_body(x_ref, w_ref, b_ref, wsrc_ref, bsrc_ref, wtgt_ref,
               btgt_ref, h0_ref, h1_ref, h2_ref, h3_ref, p_ref, self_ref):
    xb = x_ref[...]
    s = jnp.dot(xb, wsrc_ref[...].T,
                preferred_element_type=_f32) + bsrc_ref[...]
    t = jnp.dot(xb, wtgt_ref[...].T,
                preferred_element_type=_f32) + btgt_ref[...]
    es = jnp.exp(s)
    et = jnp.exp(t)
    bes = jax.lax.bitcast_convert_type(es, jnp.int32)
    bet = jax.lax.bitcast_convert_type(et, jnp.int32)
    hi = (bes + 0x8000) & jnp.int32(-65536)
    lo = jax.lax.shift_right_logical(bet + 0x8000, 16)
    p_ref[...] = hi | lo
    p = es * et
    psum = p[:, 0:1] + p[:, 1:2] + p[:, 2:3] + p[:, 3:4]
    q = p / psum
    acc = None
    for k, hk_ref in enumerate((h0_ref, h1_ref, h2_ref, h3_ref)):
        hk = jnp.dot(xb, w_ref[k * D:(k + 1) * D, :].T,
                     preferred_element_type=_f32) + b_ref[k:k + 1, :]
        hk_ref[...] = hk
        term = q[:, k:k + 1] * hk
        acc = term if acc is None else acc + term
    self_ref[...] = acc


def _prep(x, wcat, b, wsrc, bsrc, wtgt, btgt):
    BN = 400
    grid = (N // BN,)
    full = lambda shp: pl.BlockSpec(shp, lambda i: (0,) * len(shp))
    row_spec = pl.BlockSpec((BN, D), lambda i: (i, 0))
    return pl.pallas_call(
        _prep_body,
        grid=grid,
        in_specs=[
            row_spec,
            full((KD, D)), full((K, D)),
            full((K, D)), full((1, K)),
            full((K, D)), full((1, K)),
        ],
        out_specs=[
            row_spec, row_spec, row_spec, row_spec,
            pl.BlockSpec((BN, K), lambda i: (i, 0)),
            row_spec,
        ],
        out_shape=[
            jax.ShapeDtypeStruct((N, D), _f32),
            jax.ShapeDtypeStruct((N, D), _f32),
            jax.ShapeDtypeStruct((N, D), _f32),
            jax.ShapeDtypeStruct((N, D), _f32),
            jax.ShapeDtypeStruct((N, K), jnp.int32),
            jax.ShapeDtypeStruct((N, D), _f32),
        ],
    )(x, wcat, b, wsrc, bsrc, wtgt, btgt)


def _sc_body(dst_hbm, src_hbm, h0_hbm, h1_hbm, h2_hbm, h3_hbm, p_hbm,
             self_hbm, out_hbm,
             ebuf, sdst, dsti2, srcc2, dsti_s, srcc_s,
             rows2, ptab, msg, qbuf, cntv, rowids, num_sp, cnt_sp,
             sem_h, sem_i):
    cid = lax.axis_index("c")
    sid = lax.axis_index("s")
    lo = cid * NHALF
    hi_b = lo + NHALF
    eb = sid * SL
    h_hbms = (h0_hbm, h1_hbm, h2_hbm, h3_hbm)

    pltpu.sync_copy(p_hbm, ptab)

    z16f = jnp.zeros((16,), _f32)
    z16i = jnp.zeros((16,), jnp.int32)
    iota16 = lax.iota(jnp.int32, 16)

    def _zc(i, carry):
        cntv[i // 8, pl.ds((i % 8) * 16, 16)] = z16f
        return carry
    lax.fori_loop(0, CROWS * 8, _zc, 0)

    def _zbufs(i, carry):
        ebuf[pl.ds(i * 16, 16)] = z16i
        return carry
    lax.fori_loop(0, CAP // 16, _zbufs, 0)

    def _zr(i, carry):
        rows2[0, i // 8, pl.ds((i % 8) * 16, 16)] = z16f
        return carry
    lax.fori_loop(0, 64, _zr, 0)

    def _ri(i, carry):
        rowids[pl.ds(i * 16, 16)] = iota16 + i * 16
        return carry
    lax.fori_loop(0, CROWS // 16, _ri, 0)

    def _zn(i, carry):
        pltpu.sync_copy(rows2.at[0, pl.ds(0, 8)],
                        num_sp.at[pl.ds(sid * RPT + i * 8, 8)])
        return carry
    lax.fori_loop(0, RPT // 8, _zn, 0)

    @pl.when(sid < CROWS // 8)
    def _zcsp():
        pltpu.sync_copy(rows2.at[0, pl.ds(0, 8)],
                        cnt_sp.at[pl.ds(sid * 8, 8)])

    plsc.subcore_barrier()

    lov = jnp.full((16,), 0, jnp.int32) + lo
    hiv = jnp.full((16,), 0, jnp.int32) + hi_b

    def _scan(ci, nval):
        pltpu.sync_copy(dst_hbm.at[pl.ds(eb + ci * SCN, SCN)], sdst)

        def _inner(j, nv):
            dv = sdst[pl.ds(j * 16, 16)]
            m = (dv >= lov) & (dv < hiv)
            pos = (eb + ci * SCN + j * 16) + iota16
            plsc.store_compressed(ebuf.at[pl.ds(nv, 16)], pos, mask=m)
            c16 = plsc.all_reduce_population_count(m)
            return jnp.minimum(nv + c16[0], CAP - 128)
        return lax.fori_loop(0, SCN // 16, _inner, nval)

    nval = lax.fori_loop(0, SL // SCN, _scan, jnp.int32(0))

    himask = jnp.full((16,), -65536, jnp.int32)
    ones16 = jnp.ones((16,), _f32)
    nvalv = jnp.full((16,), 0, jnp.int32) + nval
    nh1 = jnp.full((16,), NHALF - 1, jnp.int32)

    def _chunk(i, carry):
        off = i * C
        p = i & 1
        pltpu.make_async_copy(dst_hbm.at[pl.ds(0, C)],
                              dsti2.at[p], sem_i).wait()
        pltpu.make_async_copy(src_hbm.at[pl.ds(0, C)],
                              srcc2.at[p], sem_i).wait()
        idxnx = ebuf.at[pl.ds((i + 1) * C, C)]
        pltpu.async_copy(dst_hbm.at[idxnx], dsti2.at[1 - p], sem_i)
        pltpu.async_copy(src_hbm.at[idxnx], srcc2.at[1 - p], sem_i)

        def _loc(b, carry2):
            dv = dsti2[p, pl.ds(b * 16, 16)] - lov
            dsti_s[pl.ds(b * 16, 16)] = jnp.minimum(jnp.maximum(dv, 0), nh1)
            srcc_s[pl.ds(b * 16, 16)] = srcc2[p, pl.ds(b * 16, 16)]
            return carry2
        lax.fori_loop(0, C // 16, _loc, 0)

        def _qblk(blk, carry2):
            dstv = dsti_s[pl.ds(blk * 16, 16)]
            srcv = srcc_s[pl.ds(blk * 16, 16)]
            valid = (off + blk * 16 + iota16) < nvalv
            plsc.addupdate_scatter(cntv, [dstv >> 7, dstv & 127], ones16,
                                   mask=valid)
            dstg4 = (dstv + lov) * 4
            srcg4 = srcv * 4
            pr = []
            for k in range(K):
                gd = plsc.load_gather(ptab, [dstg4 + k])
                gs = plsc.load_gather(ptab, [srcg4 + k])
                esv = plsc.bitcast(gd & himask, _f32)
                etv = plsc.bitcast(lax.shift_left(gs, 16), _f32)
                pr.append(esv * etv)
            rden = jnp.where(valid, 1.0 / (pr[0] + pr[1] + pr[2] + pr[3]),
                             0.0)
            for k in range(K):
                qbuf[k, pl.ds(blk * 16, 16)] = pr[k] * rden
            return carry2
        lax.fori_loop(0, C // 16, _qblk, 0)

        cp_h = pltpu.async_copy(h_hbms[0].at[srcc_s], rows2.at[0], sem_h)
        for k in range(K):
            cp_h.wait()
            if k < K - 1:
                cp_h = pltpu.async_copy(h_hbms[k + 1].at[srcc_s],
                                        rows2.at[(k + 1) & 1], sem_h)

            def _mblk(blk, carry2, k=k):
                qv = qbuf[k, pl.ds(blk * 16, 16)]
                for l in range(16):
                    e = blk * 16 + l
                    qb = jnp.full((16,), qv[l], _f32)
                    for j in range(D // 16):
                        term = qb * rows2[k & 1, e, pl.ds(j * 16, 16)]
                        if k == 0:
                            msg[e, pl.ds(j * 16, 16)] = term
                        else:
                            msg[e, pl.ds(j * 16, 16)] = (
                                msg[e, pl.ds(j * 16, 16)] + term)
                return carry2
            lax.fori_loop(0, C // 16, _mblk, 0)

        pltpu.sync_copy(msg, num_sp.at[dsti_s], add=True)
        return carry

    pltpu.async_copy(dst_hbm.at[ebuf.at[pl.ds(0, C)]], dsti2.at[0], sem_i)
    pltpu.async_copy(src_hbm.at[ebuf.at[pl.ds(0, C)]], srcc2.at[0], sem_i)
    nchunk = (nval + (C - 1)) // C
    lax.fori_loop(0, nchunk, _chunk, 0)
    pf = (nchunk & 1)
    pltpu.make_async_copy(dst_hbm.at[pl.ds(0, C)], dsti2.at[pf], sem_i).wait()
    pltpu.make_async_copy(src_hbm.at[pl.ds(0, C)], srcc2.at[pf], sem_i).wait()

    pltpu.sync_copy(cntv, cnt_sp.at[rowids], add=True)
    plsc.subcore_barrier()

    pltpu.sync_copy(cnt_sp, cntv)

    def _fin(i, carry):
        base = sid * RPT + i * 16
        pltpu.sync_copy(num_sp.at[pl.ds(base, 16)],
                        rows2.at[0, pl.ds(0, 16)])
        pltpu.sync_copy(self_hbm.at[cid, pl.ds(base, 16)],
                        rows2.at[0, pl.ds(16, 16)])
        cv = cntv[base >> 7, pl.ds(base & 127, 16)]
        rv = 1.0 / (cv + 1.0)
        for l in range(16):
            sc = jnp.full((16,), rv[l], _f32)
            for j in range(D // 16):
                msg[l, pl.ds(j * 16, 16)] = sc * (
                    rows2[0, l, pl.ds(j * 16, 16)]
                    + rows2[0, 16 + l, pl.ds(j * 16, 16)])
        pltpu.sync_copy(msg.at[pl.ds(0, 16)],
                        out_hbm.at[cid, pl.ds(base, 16)])
        return carry
    lax.fori_loop(0, RPT // 16, _fin, 0)


def _sc_edges(dst, src, h0, h1, h2, h3, ptab, selfp):
    mesh = plsc.VectorSubcoreMesh(core_axis_name="c", subcore_axis_name="s",
                                  num_cores=NC, num_subcores=NS)
    f = pl.kernel(
        _sc_body,
        out_type=jax.ShapeDtypeStruct((NC, NH, D), _f32),
        mesh=mesh,
        compiler_params=pltpu.CompilerParams(needs_layout_passes=False),
        scratch_types=[
            pltpu.VMEM((CAP,), jnp.int32),
            pltpu.VMEM((SCN,), jnp.int32),
            pltpu.VMEM((2, C), jnp.int32),
            pltpu.VMEM((2, C), jnp.int32),
            pltpu.VMEM((C,), jnp.int32),
            pltpu.VMEM((C,), jnp.int32),
            pltpu.VMEM((2, C, D), _f32),
            pltpu.VMEM((N * K,), jnp.int32),
            pltpu.VMEM((C, D), _f32),
            pltpu.VMEM((K, C), _f32),
            pltpu.VMEM((CROWS, 128), _f32),
            pltpu.VMEM((CROWS,), jnp.int32),
            pltpu.VMEM_SHARED((NH, D), _f32),
            pltpu.VMEM_SHARED((CROWS, 128), _f32),
            pltpu.SemaphoreType.DMA,
            pltpu.SemaphoreType.DMA,
        ],
    )
    return f(dst, src, h0, h1, h2, h3, ptab, selfp)


@jax.jit
def kernel(x, edge_index, W, b, Wsrc, bsrc, Wtgt, btgt):
    wcat = W.reshape(KD, D)
    h0, h1, h2, h3, ptab, selfmsg = _prep(x, wcat, b,
                                          Wsrc, bsrc.reshape(1, K),
                                          Wtgt, btgt.reshape(1, K))
    dst = edge_index[0]
    src = edge_index[1]
    selfp = jnp.zeros((NC, NH, D), _f32)
    selfp = selfp.at[0, :NHALF].set(selfmsg[:NHALF])
    selfp = selfp.at[1, :NHALF].set(selfmsg[NHALF:])
    o = _sc_edges(dst, src, h0, h1, h2, h3, ptab.reshape(N * K), selfp)
    return jnp.concatenate([o[0, :NHALF], o[1, :NHALF]], axis=0)
